# baseline (speedup 1.0000x reference)
import jax
import jax.numpy as jnp
from jax import lax
from jax.experimental import pallas as pl
from jax.experimental.pallas import tpu as pltpu

B = 16
H = 2048
D_OBS = 64
D_ACT = 16
HB = 512
NH = H // HB


def _body(seq_smem, seq_row, seq_col, drop_ref,
          obs_in, act_in, nobs_in, rew_in, done_in,
          obs_out, act_out, nobs_out, rew_out, done_out,
          attn_out, omask_out, amask_out):
    h = pl.program_id(0)
    b = pl.program_id(1)

    rows = h * HB + lax.broadcasted_iota(jnp.int32, (HB, 1), 0)

    lens_b = seq_smem[b]
    valid_col = rows < lens_b
    obs_out[:, 0, 0, :] = jnp.where(valid_col, obs_in[0], 0.0)
    act_out[:, 0, 0, :] = jnp.where(valid_col, act_in[0], 0.0)
    nobs_out[:, 0, 0, :] = jnp.where(valid_col, nobs_in[0], 0.0)

    @pl.when(b == 0)
    def _():
        valid_bt = rows < seq_row[0:1, :]
        rew_out[:, :] = jnp.where(valid_bt, rew_in[:, :].T, 0.0)
        done_out[:, :] = jnp.where(valid_bt, done_in[:, :].T, 0.0)
        col = lax.broadcasted_iota(jnp.int32, (HB, H), 1)
        attn_out[:, :] = jnp.where(col > rows, -jnp.inf, 0.0).astype(jnp.float32)

    @pl.when((b == 0) & (h == 0))
    def _():
        pos = lax.broadcasted_iota(jnp.int32, (B, H), 1)
        pad = pos >= seq_col[:, :]
        omask_out[:, :] = (pad | (drop_ref[:, :] != 0)).astype(jnp.int8)
        amask_out[:, :] = pad.astype(jnp.int8)


def _drop_const():
    kd = jax.random.key(1)
    ka, kb, _, _ = jax.random.split(kd, 4)
    full_obs = jax.random.uniform(ka, (B,))
    r_obs = jax.random.uniform(kb, (B, H))
    pos = jnp.arange(H)
    drop = (full_obs[:, None] >= 0.5) & (pos[None, :] > 0) & (r_obs < 0.5)
    return drop.astype(jnp.int8)


def kernel(obs_flat, act_flat, next_obs_flat, rew_flat, done_flat, seqlens):
    obs3 = obs_flat.reshape(B, H, D_OBS)
    act3 = act_flat.reshape(B, H, D_ACT)
    nobs3 = next_obs_flat.reshape(B, H, D_OBS)
    rew2 = rew_flat.reshape(B, H)
    done2 = done_flat.reshape(B, H)
    seq_row = seqlens.reshape(1, B)
    seq_col = seqlens.reshape(B, 1)
    drop = _drop_const()

    out_shapes = (
        jax.ShapeDtypeStruct((H, B, 1, D_OBS), jnp.float32),
        jax.ShapeDtypeStruct((H, B, 1, D_ACT), jnp.float32),
        jax.ShapeDtypeStruct((H, B, 1, D_OBS), jnp.float32),
        jax.ShapeDtypeStruct((H, B), jnp.float32),
        jax.ShapeDtypeStruct((H, B), jnp.float32),
        jax.ShapeDtypeStruct((H, H), jnp.float32),
        jax.ShapeDtypeStruct((B, H), jnp.int8),
        jax.ShapeDtypeStruct((B, H), jnp.int8),
    )
    in_specs = [
        pl.BlockSpec(memory_space=pltpu.SMEM),
        pl.BlockSpec((1, B), lambda h, b: (0, 0)),
        pl.BlockSpec((B, 1), lambda h, b: (0, 0)),
        pl.BlockSpec((B, H), lambda h, b: (0, 0)),
        pl.BlockSpec((1, HB, D_OBS), lambda h, b: (b, h, 0)),
        pl.BlockSpec((1, HB, D_ACT), lambda h, b: (b, h, 0)),
        pl.BlockSpec((1, HB, D_OBS), lambda h, b: (b, h, 0)),
        pl.BlockSpec((B, HB), lambda h, b: (0, h)),
        pl.BlockSpec((B, HB), lambda h, b: (0, h)),
    ]
    out_specs = [
        pl.BlockSpec((HB, 1, 1, D_OBS), lambda h, b: (h, b, 0, 0)),
        pl.BlockSpec((HB, 1, 1, D_ACT), lambda h, b: (h, b, 0, 0)),
        pl.BlockSpec((HB, 1, 1, D_OBS), lambda h, b: (h, b, 0, 0)),
        pl.BlockSpec((HB, B), lambda h, b: (h, 0)),
        pl.BlockSpec((HB, B), lambda h, b: (h, 0)),
        pl.BlockSpec((HB, H), lambda h, b: (h, 0)),
        pl.BlockSpec((B, H), lambda h, b: (0, 0)),
        pl.BlockSpec((B, H), lambda h, b: (0, 0)),
    ]
    outs = pl.pallas_call(
        _body,
        grid=(NH, B),
        in_specs=in_specs,
        out_specs=out_specs,
        out_shape=out_shapes,
    )(seqlens, seq_row, seq_col, drop, obs3, act3, nobs3, rew2, done2)

    obs4, act4, nobs4, rew, done, attn, omask, amask = outs
    obs = obs4.reshape(H, B, D_OBS)
    act = act4.reshape(H, B, D_ACT)
    next_obs = nobs4.reshape(H, B, D_OBS)
    return (obs, act, next_obs, rew, done, seqlens,
            attn, attn, omask.astype(jnp.bool_), amask.astype(jnp.bool_))


# speedup vs baseline: 2.3224x; 2.3224x over previous
import jax
import jax.numpy as jnp
from jax import lax
from jax.experimental import pallas as pl
from jax.experimental.pallas import tpu as pltpu

B = 16
H = 2048
D_OBS = 64
D_ACT = 16
HB = 512
NH = H // HB


def _body(seq_smem, seq_col, drop_ref,
          xo_ref, xa_ref, xn_ref, rew_ref, done_ref,
          obs_out, act_out, nobs_out, rew_out, done_out,
          attn1_out, attn2_out, omask_out, amask_out):
    h = pl.program_id(0)
    b = pl.program_id(1)
    lens_b = seq_smem[b]

    t_do = h * HB + lax.broadcasted_iota(jnp.int32, (D_OBS, HB), 1)
    obs_out[0] = jnp.where(t_do < lens_b, xo_ref[...], 0.0)
    nobs_out[0] = jnp.where(t_do < lens_b, xn_ref[...], 0.0)
    t_da = h * HB + lax.broadcasted_iota(jnp.int32, (D_ACT, HB), 1)
    act_out[0] = jnp.where(t_da < lens_b, xa_ref[...], 0.0)

    @pl.when(b == 0)
    def _():
        t_bt = h * HB + lax.broadcasted_iota(jnp.int32, (B, HB), 1)
        valid = t_bt < seq_col[...]
        rew_out[...] = jnp.where(valid, rew_ref[...], 0.0)
        done_out[...] = jnp.where(valid, done_ref[...], 0.0)
        r = h * HB + lax.broadcasted_iota(jnp.int32, (HB, H), 0)
        c = lax.broadcasted_iota(jnp.int32, (HB, H), 1)
        a = jnp.where(c > r, -jnp.inf, 0.0).astype(jnp.float32)
        attn1_out[...] = a
        attn2_out[...] = a

    @pl.when((b == 0) & (h == 0))
    def _():
        pos = lax.broadcasted_iota(jnp.int32, (B, H), 1)
        pad = pos >= seq_col[...]
        omask_out[...] = pad | (drop_ref[...] != 0)
        amask_out[...] = pad


def _drop_const():
    kd = jax.random.key(1)
    ka, kb, _, _ = jax.random.split(kd, 4)
    full_obs = jax.random.uniform(ka, (B,))
    r_obs = jax.random.uniform(kb, (B, H))
    pos = jnp.arange(H)
    drop = (full_obs[:, None] >= 0.5) & (pos[None, :] > 0) & (r_obs < 0.5)
    return drop.astype(jnp.int8)


def kernel(obs_flat, act_flat, next_obs_flat, rew_flat, done_flat, seqlens):
    xo = obs_flat.T
    xa = act_flat.T
    xn = next_obs_flat.T
    rew2 = rew_flat.reshape(B, H)
    done2 = done_flat.reshape(B, H)
    seq_col = seqlens.reshape(B, 1)
    drop = _drop_const()

    out_shapes = (
        jax.ShapeDtypeStruct((B, D_OBS, H), jnp.float32),
        jax.ShapeDtypeStruct((B, D_ACT, H), jnp.float32),
        jax.ShapeDtypeStruct((B, D_OBS, H), jnp.float32),
        jax.ShapeDtypeStruct((B, H), jnp.float32),
        jax.ShapeDtypeStruct((B, H), jnp.float32),
        jax.ShapeDtypeStruct((H, H), jnp.float32),
        jax.ShapeDtypeStruct((H, H), jnp.float32),
        jax.ShapeDtypeStruct((B, H), jnp.bool_),
        jax.ShapeDtypeStruct((B, H), jnp.bool_),
    )
    in_specs = [
        pl.BlockSpec(memory_space=pltpu.SMEM),
        pl.BlockSpec((B, 1), lambda h, b: (0, 0)),
        pl.BlockSpec((B, H), lambda h, b: (0, 0)),
        pl.BlockSpec((D_OBS, HB), lambda h, b: (0, b * NH + h)),
        pl.BlockSpec((D_ACT, HB), lambda h, b: (0, b * NH + h)),
        pl.BlockSpec((D_OBS, HB), lambda h, b: (0, b * NH + h)),
        pl.BlockSpec((B, HB), lambda h, b: (0, h)),
        pl.BlockSpec((B, HB), lambda h, b: (0, h)),
    ]
    out_specs = [
        pl.BlockSpec((1, D_OBS, HB), lambda h, b: (b, 0, h)),
        pl.BlockSpec((1, D_ACT, HB), lambda h, b: (b, 0, h)),
        pl.BlockSpec((1, D_OBS, HB), lambda h, b: (b, 0, h)),
        pl.BlockSpec((B, HB), lambda h, b: (0, h)),
        pl.BlockSpec((B, HB), lambda h, b: (0, h)),
        pl.BlockSpec((HB, H), lambda h, b: (h, 0)),
        pl.BlockSpec((HB, H), lambda h, b: (h, 0)),
        pl.BlockSpec((B, H), lambda h, b: (0, 0)),
        pl.BlockSpec((B, H), lambda h, b: (0, 0)),
    ]
    outs = pl.pallas_call(
        _body,
        grid=(NH, B),
        in_specs=in_specs,
        out_specs=out_specs,
        out_shape=out_shapes,
    )(seqlens, seq_col, drop, xo, xa, xn, rew2, done2)

    obs_p, act_p, nobs_p, rew_p, done_p, attn1, attn2, omask, amask = outs
    obs = jnp.transpose(obs_p, (2, 0, 1))
    act = jnp.transpose(act_p, (2, 0, 1))
    next_obs = jnp.transpose(nobs_p, (2, 0, 1))
    rew = rew_p.T
    done = done_p.T
    return (obs, act, next_obs, rew, done, seqlens,
            attn1, attn2, omask, amask)


# speedup vs baseline: 2.4762x; 1.0662x over previous
import functools

import jax
import jax.numpy as jnp
from jax import lax
from jax.experimental import pallas as pl
from jax.experimental.pallas import tpu as pltpu
from jax.experimental.pallas import tpu_sc as plsc

B = 16
H = 2048
D_OBS = 64
D_ACT = 16
HB = 512
NH = H // HB
W = 1024
NT = W // 128


def _sc_unit(src, dst, b, g, t0, n, buf, zbuf, lane16, sem):
    nloc = jnp.clip(n - t0, 0, W)
    nf = nloc // 128
    bnd = nloc - nf * 128
    pltpu.sync_copy(src.at[pl.ds(8 * g, 8), pl.ds(b * H + t0, W)], buf)

    @pl.when(bnd > 0)
    def _():
        def _row(r, _):
            def _grp(j, _):
                sl = pl.ds(nf * 128 + j * 16, 16)
                col = nf * 128 + j * 16 + lane16
                buf[r, sl] = jnp.where(col < nloc, buf[r, sl], 0.0)
                return 0
            return lax.fori_loop(0, 8, _grp, 0)
        lax.fori_loop(0, 8, _row, 0)

    npref = nf + jnp.where(bnd > 0, 1, 0)
    for k in (8, 4, 2, 1):
        st = npref & (~(2 * k - 1) & 0xF)

        @pl.when((npref & k) != 0)
        def _(st=st, k=k):
            pltpu.sync_copy(buf.at[:, pl.ds(st * 128, k * 128)],
                            dst.at[b, pl.ds(8 * g, 8),
                                   pl.ds(t0 + st * 128, k * 128)])

    zc = NT - npref
    for k in (8, 4, 2, 1):
        zo = zc & (~(2 * k - 1) & 0xF)

        @pl.when((zc & k) != 0)
        def _(zo=zo, k=k):
            pltpu.sync_copy(zbuf.at[:, pl.ds(0, k * 128)],
                            dst.at[b, pl.ds(8 * g, 8),
                                   pl.ds(t0 + (npref + zo) * 128, k * 128)])


def _make_sc_kernel():
    mesh = plsc.VectorSubcoreMesh(core_axis_name="c", subcore_axis_name="s")

    @functools.partial(
        pl.kernel,
        out_type=(jax.ShapeDtypeStruct((B, D_OBS, H), jnp.float32),
                  jax.ShapeDtypeStruct((B, D_ACT, H), jnp.float32),
                  jax.ShapeDtypeStruct((B, D_OBS, H), jnp.float32)),
        mesh=mesh,
        scratch_types=[pltpu.VMEM((8, W), jnp.float32),
                       pltpu.VMEM((8, W), jnp.float32),
                       pltpu.VMEM((B,), jnp.int32),
                       pltpu.SemaphoreType.DMA],
        compiler_params=pltpu.CompilerParams(use_tc_tiling_on_sc=True,
                                             needs_layout_passes=False),
    )
    def sc_kernel(xo_hbm, xa_hbm, xn_hbm, seq_hbm, zsrc_hbm,
                  obs_hbm, act_hbm, nobs_hbm, buf, zbuf, lens_v, sem):
        w = lax.axis_index("s") * 2 + lax.axis_index("c")
        b = w // 2
        t0 = (w % 2) * W
        pltpu.sync_copy(seq_hbm, lens_v)
        pltpu.sync_copy(zsrc_hbm, zbuf)
        idx16 = lax.iota(jnp.int32, 16)
        lane16 = lax.iota(jnp.int32, 16)
        n = jnp.max(jnp.where(idx16 == b, lens_v[...], 0))
        for g in range(D_OBS // 8):
            _sc_unit(xo_hbm, obs_hbm, b, g, t0, n, buf, zbuf, lane16, sem)
        for g in range(D_OBS // 8):
            _sc_unit(xn_hbm, nobs_hbm, b, g, t0, n, buf, zbuf, lane16, sem)
        for g in range(D_ACT // 8):
            _sc_unit(xa_hbm, act_hbm, b, g, t0, n, buf, zbuf, lane16, sem)

    return sc_kernel


_SC_KERNEL = _make_sc_kernel()


def _tc_body(seq_col, drop_ref, rew_ref, done_ref,
             rew_out, done_out, attn1_out, attn2_out, omask_out, amask_out):
    h = pl.program_id(0)
    t_bt = h * HB + lax.broadcasted_iota(jnp.int32, (B, HB), 1)
    valid = t_bt < seq_col[...]
    rew_out[...] = jnp.where(valid, rew_ref[...], 0.0)
    done_out[...] = jnp.where(valid, done_ref[...], 0.0)
    r = h * HB + lax.broadcasted_iota(jnp.int32, (HB, H), 0)
    c = lax.broadcasted_iota(jnp.int32, (HB, H), 1)
    a = jnp.where(c > r, -jnp.inf, 0.0).astype(jnp.float32)
    attn1_out[...] = a
    attn2_out[...] = a

    @pl.when(h == 0)
    def _():
        pos = lax.broadcasted_iota(jnp.int32, (B, H), 1)
        pad = pos >= seq_col[...]
        omask_out[...] = pad | (drop_ref[...] != 0)
        amask_out[...] = pad


def _drop_const():
    kd = jax.random.key(1)
    ka, kb, _, _ = jax.random.split(kd, 4)
    full_obs = jax.random.uniform(ka, (B,))
    r_obs = jax.random.uniform(kb, (B, H))
    pos = jnp.arange(H)
    drop = (full_obs[:, None] >= 0.5) & (pos[None, :] > 0) & (r_obs < 0.5)
    return drop.astype(jnp.int8)


def kernel(obs_flat, act_flat, next_obs_flat, rew_flat, done_flat, seqlens):
    xo = obs_flat.T
    xa = act_flat.T
    xn = next_obs_flat.T
    rew2 = rew_flat.reshape(B, H)
    done2 = done_flat.reshape(B, H)
    seq_col = seqlens.reshape(B, 1)
    drop = _drop_const()
    zsrc = jnp.zeros((8, W), jnp.float32)

    obs_p, act_p, nobs_p = _SC_KERNEL(xo, xa, xn, seqlens, zsrc)

    tc_out_shapes = (
        jax.ShapeDtypeStruct((B, H), jnp.float32),
        jax.ShapeDtypeStruct((B, H), jnp.float32),
        jax.ShapeDtypeStruct((H, H), jnp.float32),
        jax.ShapeDtypeStruct((H, H), jnp.float32),
        jax.ShapeDtypeStruct((B, H), jnp.bool_),
        jax.ShapeDtypeStruct((B, H), jnp.bool_),
    )
    tc_in_specs = [
        pl.BlockSpec((B, 1), lambda h: (0, 0)),
        pl.BlockSpec((B, H), lambda h: (0, 0)),
        pl.BlockSpec((B, HB), lambda h: (0, h)),
        pl.BlockSpec((B, HB), lambda h: (0, h)),
    ]
    tc_out_specs = [
        pl.BlockSpec((B, HB), lambda h: (0, h)),
        pl.BlockSpec((B, HB), lambda h: (0, h)),
        pl.BlockSpec((HB, H), lambda h: (h, 0)),
        pl.BlockSpec((HB, H), lambda h: (h, 0)),
        pl.BlockSpec((B, H), lambda h: (0, 0)),
        pl.BlockSpec((B, H), lambda h: (0, 0)),
    ]
    rew_p, done_p, attn1, attn2, omask, amask = pl.pallas_call(
        _tc_body,
        grid=(NH,),
        in_specs=tc_in_specs,
        out_specs=tc_out_specs,
        out_shape=tc_out_shapes,
    )(seq_col, drop, rew2, done2)

    obs = jnp.transpose(obs_p, (2, 0, 1))
    act = jnp.transpose(act_p, (2, 0, 1))
    next_obs = jnp.transpose(nobs_p, (2, 0, 1))
    rew = rew_p.T
    done = done_p.T
    return (obs, act, next_obs, rew, done, seqlens,
            attn1, attn2, omask, amask)
